# baseline (speedup 1.0000x reference)
import functools

import jax
import jax.numpy as jnp
from jax.experimental import pallas as pl
from jax import lax
from jax.experimental.pallas import tpu as pltpu
from jax.experimental.pallas import tpu_sc as plsc

DIM = 256
NUM_HEADS = 8
HEAD_DIM = DIM // NUM_HEADS
GS = 128
GB = 4
QC = 64
LOG2E = 1.4426950408889634


def _attn_body(scale_ref, qkv_ref, w_ref, b_ref, out_ref):
    scale = jnp.exp(jnp.minimum(scale_ref[0, 0], jnp.log(1.0 / 0.01)))
    sl2 = scale * LOG2E
    wb = w_ref[...].astype(jnp.bfloat16)
    bias = b_ref[...]
    for g in range(GB):
        blk = qkv_ref[g]
        q = blk[:, :DIM]
        k = blk[:, DIM:2 * DIM]
        v = blk[:, 2 * DIM:]
        outs = []
        for h in range(NUM_HEADS):
            sl = slice(h * HEAD_DIM, (h + 1) * HEAD_DIM)
            kh = k[:, sl]
            vh = v[:, sl].astype(jnp.bfloat16)
            accs = []
            for qc in range(GS // QC):
                qh = q[qc * QC:(qc + 1) * QC, sl]
                s = jax.lax.dot_general(qh, kh, (((1,), (1,)), ((), ())),
                                        preferred_element_type=jnp.float32) * sl2
                m = jnp.max(s, axis=-1, keepdims=True)
                e = jnp.exp2(s - m)
                r = 1.0 / jnp.sum(e, axis=-1, keepdims=True)
                acc = jax.lax.dot_general(e.astype(jnp.bfloat16), vh,
                                          (((1,), (0,)), ((), ())),
                                          preferred_element_type=jnp.float32)
                accs.append(acc * r)
            outs.append(jnp.concatenate(accs, axis=0))
        o = jnp.concatenate(outs, axis=1)
        res = jax.lax.dot_general(o.astype(jnp.bfloat16), wb,
                                  (((1,), (1,)), ((), ())),
                                  preferred_element_type=jnp.float32)
        out_ref[g] = res + bias


def _grouped_attention(grouped, proj_w, proj_b, logit_scale):
    nblk = grouped.shape[0]
    return pl.pallas_call(
        _attn_body,
        grid=(nblk // GB,),
        in_specs=[
            pl.BlockSpec(memory_space=pltpu.SMEM),
            pl.BlockSpec((GB, GS, 3 * DIM), lambda i: (i, 0, 0)),
            pl.BlockSpec((DIM, DIM), lambda i: (0, 0)),
            pl.BlockSpec((1, DIM), lambda i: (0, 0)),
        ],
        out_specs=pl.BlockSpec((GB, GS, DIM), lambda i: (i, 0, 0)),
        out_shape=jax.ShapeDtypeStruct((nblk, GS, DIM), jnp.float32),
        compiler_params=pltpu.CompilerParams(
            dimension_semantics=("arbitrary",)),
    )(logit_scale, grouped, proj_w, proj_b.reshape(1, DIM))


SC_B = 2
SC_N = 16384
SC_M = 64
NSUB = 16
TPW = SC_N // NSUB
CH = 256
NVR = TPW // 16

RANK_BASE = 1


def _routing_body(sim_hbm, sortidx_hbm, rank_hbm,
                  simv, idsv, posv, pos2v, valv, zerov, histv, runv, allhv,
                  histgrid, sortshared, sem):
    c = lax.axis_index("c")
    s = lax.axis_index("s")
    base = s * TPW

    def chunk_body(ci, _):
        pltpu.sync_copy(sim_hbm.at[c, pl.ds((base + ci * CH) * SC_M, CH * SC_M)], simv)

        def tok_body(t0, _):
            rowbase = (lax.iota(jnp.int32, 16) + t0 * 16) * SC_M
            init = (jnp.full((16,), -jnp.inf, jnp.float32),
                    jnp.zeros((16,), jnp.int32))

            def m_body(m, carry):
                best, bid = carry
                val = plsc.load_gather(simv, [rowbase + m])
                upd = val > best
                return (jnp.where(upd, val, best), jnp.where(upd, m, bid))

            best, bid = lax.fori_loop(0, SC_M, m_body, init)
            idsv[pl.ds(ci * CH + t0 * 16, 16)] = bid
            return _

        return lax.fori_loop(0, CH // 16, tok_body, _)

    lax.fori_loop(0, TPW // CH, chunk_body, None)

    for j in range(4):
        histv[pl.ds(j * 16, 16)] = jnp.zeros((16,), jnp.int32)

    def h_body(i, _):
        ids = idsv[pl.ds(i * 16, 16)]
        r, last = plsc.scan_count(ids)
        r = r - RANK_BASE
        cur = plsc.load_gather(histv, [ids])
        plsc.store_scatter(histv, [ids], cur + r + 1, mask=last)
        return _

    lax.fori_loop(0, NVR, h_body, None)

    pltpu.sync_copy(histv, histgrid.at[s])
    plsc.subcore_barrier()
    pltpu.sync_copy(histgrid, allhv)

    carry = jnp.zeros((), jnp.int32)
    for j in range(4):
        tot = jnp.zeros((16,), jnp.int32)
        mybefore = jnp.zeros((16,), jnp.int32)
        for sp in range(NSUB):
            h = allhv[sp, pl.ds(j * 16, 16)]
            tot = tot + h
            mybefore = mybefore + jnp.where(sp < s, h, 0)
        csum = plsc.cumsum(tot)
        excl = csum - tot + carry
        runv[pl.ds(j * 16, 16)] = excl + mybefore
        carry = carry + jnp.sum(tot)

    def p_body(i, _):
        ids = idsv[pl.ds(i * 16, 16)]
        r, last = plsc.scan_count(ids)
        r = r - RANK_BASE
        cur = plsc.load_gather(runv, [ids])
        pos = cur + r
        posv[pl.ds(i * 16, 16)] = pos
        row = i // (128 // 16)
        col = (i % (128 // 16)) * 16
        pos2v[row, pl.ds(col, 16)] = pos
        valv[pl.ds(i * 16, 16)] = lax.iota(jnp.int32, 16) + (base + i * 16)
        plsc.store_scatter(runv, [ids], cur + r + 1, mask=last)
        return _

    lax.fori_loop(0, NVR, p_body, None)

    pltpu.sync_copy(posv, rank_hbm.at[c, pl.ds(base, TPW)])

    def z_body(i, _):
        zerov[pl.ds(i * 16, 16)] = jnp.zeros((16,), jnp.int32)
        return _

    lax.fori_loop(0, NVR, z_body, None)
    pltpu.sync_copy(zerov, sortshared.at[pl.ds(s * TPW, TPW)])
    plsc.subcore_barrier()
    for j in range(TPW // 128):
        pltpu.async_copy(
            valv.at[pl.ds(j * 128, 128)],
            sortshared.at[pos2v.at[j]],
            sem,
            add=True,
        ).wait()
    plsc.subcore_barrier()
    pltpu.sync_copy(sortshared.at[pl.ds(s * TPW, TPW)],
                    sortidx_hbm.at[c, pl.ds(base, TPW)])


def make_routing():
    mesh = plsc.VectorSubcoreMesh(core_axis_name="c", subcore_axis_name="s")
    return functools.partial(
        pl.kernel,
        out_type=(jax.ShapeDtypeStruct((SC_B, SC_N), jnp.int32),
                  jax.ShapeDtypeStruct((SC_B, SC_N), jnp.int32)),
        mesh=mesh,
        compiler_params=pltpu.CompilerParams(needs_layout_passes=False),
        scratch_types=[
            pltpu.VMEM((CH * SC_M,), jnp.float32),
            pltpu.VMEM((TPW,), jnp.int32),
            pltpu.VMEM((TPW,), jnp.int32),
            pltpu.VMEM((TPW // 128, 128), jnp.int32),
            pltpu.VMEM((TPW,), jnp.int32),
            pltpu.VMEM((TPW,), jnp.int32),
            pltpu.VMEM((SC_M,), jnp.int32),
            pltpu.VMEM((SC_M,), jnp.int32),
            pltpu.VMEM((NSUB, SC_M), jnp.int32),
            pltpu.VMEM_SHARED((NSUB, SC_M), jnp.int32),
            pltpu.VMEM_SHARED((SC_N,), jnp.int32),
            pltpu.SemaphoreType.DMA,
        ],
    )(_routing_body)


def kernel(qkv, sim, x_size, proj_w, proj_b, logit_scale):
    b, n, c3 = qkv.shape
    ng = n // GS
    routing = make_routing()
    sort_idx, rank = routing(sim.reshape(b, n * SC_M))
    shuffled = jnp.take_along_axis(qkv, sort_idx[:, :, None], axis=1)
    grouped = shuffled.reshape(b * ng, GS, c3)
    out = _grouped_attention(grouped, proj_w, proj_b, logit_scale)
    out = out.reshape(b, n, DIM)
    x = jnp.take_along_axis(out, rank[:, :, None], axis=1)
    return x


# speedup vs baseline: 1.6624x; 1.6624x over previous
import functools

import jax
import jax.numpy as jnp
from jax.experimental import pallas as pl
from jax import lax
from jax.experimental.pallas import tpu as pltpu
from jax.experimental.pallas import tpu_sc as plsc

DIM = 256
NUM_HEADS = 8
HEAD_DIM = DIM // NUM_HEADS
GS = 128
GB = 8
LOG2E = 1.4426950408889634


def _attn_body(scale_ref, qkv_ref, w_ref, b_ref, out_ref):
    scale = jnp.exp(jnp.minimum(scale_ref[0, 0], jnp.log(1.0 / 0.01)))
    sl2 = scale * LOG2E
    wb = w_ref[...].astype(jnp.bfloat16)
    bias = b_ref[...]
    for g in range(GB):
        blk = qkv_ref[g]
        q = blk[:, :DIM]
        k = blk[:, DIM:2 * DIM]
        v = blk[:, 2 * DIM:]
        outs = []
        for h in range(NUM_HEADS):
            sl = slice(h * HEAD_DIM, (h + 1) * HEAD_DIM)
            s = jax.lax.dot_general(q[:, sl], k[:, sl], (((1,), (1,)), ((), ())),
                                    preferred_element_type=jnp.float32) * sl2
            m = jnp.max(s, axis=-1, keepdims=True)
            e = jnp.exp2(s - m)
            r = 1.0 / jnp.sum(e, axis=-1, keepdims=True)
            acc = jax.lax.dot_general(e.astype(jnp.bfloat16),
                                      v[:, sl].astype(jnp.bfloat16),
                                      (((1,), (0,)), ((), ())),
                                      preferred_element_type=jnp.float32)
            outs.append(acc * r)
        o = jnp.concatenate(outs, axis=1)
        res = jax.lax.dot_general(o.astype(jnp.bfloat16), wb,
                                  (((1,), (1,)), ((), ())),
                                  preferred_element_type=jnp.float32)
        out_ref[g] = res + bias


def _grouped_attention(grouped, proj_w, proj_b, logit_scale):
    nblk = grouped.shape[0]
    return pl.pallas_call(
        _attn_body,
        grid=(nblk // GB,),
        in_specs=[
            pl.BlockSpec(memory_space=pltpu.SMEM),
            pl.BlockSpec((GB, GS, 3 * DIM), lambda i: (i, 0, 0)),
            pl.BlockSpec((DIM, DIM), lambda i: (0, 0)),
            pl.BlockSpec((1, DIM), lambda i: (0, 0)),
        ],
        out_specs=pl.BlockSpec((GB, GS, DIM), lambda i: (i, 0, 0)),
        out_shape=jax.ShapeDtypeStruct((nblk, GS, DIM), jnp.float32),
        compiler_params=pltpu.CompilerParams(
            dimension_semantics=("arbitrary",)),
    )(logit_scale, grouped, proj_w, proj_b.reshape(1, DIM))


SC_B = 2
SC_N = 16384
SC_M = 64
NSUB = 16
TPW = SC_N // NSUB
CH = 256
NVR = TPW // 16

RANK_BASE = 1


def _routing_body(sim_hbm, sortidx_hbm, rank_hbm,
                  simv, idsv, posv, pos2v, valv, zerov, histv, runv, allhv,
                  histgrid, sortshared, sem):
    c = lax.axis_index("c")
    s = lax.axis_index("s")
    base = s * TPW

    def chunk_body(ci, _):
        pltpu.sync_copy(sim_hbm.at[c, pl.ds((base + ci * CH) * SC_M, CH * SC_M)], simv)

        def tok_body(t0, _):
            rowbase = (lax.iota(jnp.int32, 16) + t0 * 16) * SC_M
            init = (jnp.full((16,), -jnp.inf, jnp.float32),
                    jnp.zeros((16,), jnp.int32))

            def m_body(m, carry):
                best, bid = carry
                val = plsc.load_gather(simv, [rowbase + m])
                upd = val > best
                return (jnp.where(upd, val, best), jnp.where(upd, m, bid))

            best, bid = lax.fori_loop(0, SC_M, m_body, init)
            idsv[pl.ds(ci * CH + t0 * 16, 16)] = bid
            return _

        return lax.fori_loop(0, CH // 16, tok_body, _)

    lax.fori_loop(0, TPW // CH, chunk_body, None)

    for j in range(4):
        histv[pl.ds(j * 16, 16)] = jnp.zeros((16,), jnp.int32)

    def h_body(i, _):
        ids = idsv[pl.ds(i * 16, 16)]
        r, last = plsc.scan_count(ids)
        r = r - RANK_BASE
        cur = plsc.load_gather(histv, [ids])
        plsc.store_scatter(histv, [ids], cur + r + 1, mask=last)
        return _

    lax.fori_loop(0, NVR, h_body, None)

    pltpu.sync_copy(histv, histgrid.at[s])
    plsc.subcore_barrier()
    pltpu.sync_copy(histgrid, allhv)

    carry = jnp.zeros((), jnp.int32)
    for j in range(4):
        tot = jnp.zeros((16,), jnp.int32)
        mybefore = jnp.zeros((16,), jnp.int32)
        for sp in range(NSUB):
            h = allhv[sp, pl.ds(j * 16, 16)]
            tot = tot + h
            mybefore = mybefore + jnp.where(sp < s, h, 0)
        csum = plsc.cumsum(tot)
        excl = csum - tot + carry
        runv[pl.ds(j * 16, 16)] = excl + mybefore
        carry = carry + jnp.sum(tot)

    def p_body(i, _):
        ids = idsv[pl.ds(i * 16, 16)]
        r, last = plsc.scan_count(ids)
        r = r - RANK_BASE
        cur = plsc.load_gather(runv, [ids])
        pos = cur + r
        posv[pl.ds(i * 16, 16)] = pos
        row = i // (128 // 16)
        col = (i % (128 // 16)) * 16
        pos2v[row, pl.ds(col, 16)] = pos
        valv[pl.ds(i * 16, 16)] = lax.iota(jnp.int32, 16) + (base + i * 16)
        plsc.store_scatter(runv, [ids], cur + r + 1, mask=last)
        return _

    lax.fori_loop(0, NVR, p_body, None)

    pltpu.sync_copy(posv, rank_hbm.at[c, pl.ds(base, TPW)])

    def z_body(i, _):
        zerov[pl.ds(i * 16, 16)] = jnp.zeros((16,), jnp.int32)
        return _

    lax.fori_loop(0, NVR, z_body, None)
    pltpu.sync_copy(zerov, sortshared.at[pl.ds(s * TPW, TPW)])
    plsc.subcore_barrier()
    for j in range(TPW // 128):
        pltpu.async_copy(
            valv.at[pl.ds(j * 128, 128)],
            sortshared.at[pos2v.at[j]],
            sem,
            add=True,
        ).wait()
    plsc.subcore_barrier()
    pltpu.sync_copy(sortshared.at[pl.ds(s * TPW, TPW)],
                    sortidx_hbm.at[c, pl.ds(base, TPW)])


def make_routing():
    mesh = plsc.VectorSubcoreMesh(core_axis_name="c", subcore_axis_name="s")
    return functools.partial(
        pl.kernel,
        out_type=(jax.ShapeDtypeStruct((SC_B, SC_N), jnp.int32),
                  jax.ShapeDtypeStruct((SC_B, SC_N), jnp.int32)),
        mesh=mesh,
        compiler_params=pltpu.CompilerParams(needs_layout_passes=False),
        scratch_types=[
            pltpu.VMEM((CH * SC_M,), jnp.float32),
            pltpu.VMEM((TPW,), jnp.int32),
            pltpu.VMEM((TPW,), jnp.int32),
            pltpu.VMEM((TPW // 128, 128), jnp.int32),
            pltpu.VMEM((TPW,), jnp.int32),
            pltpu.VMEM((TPW,), jnp.int32),
            pltpu.VMEM((SC_M,), jnp.int32),
            pltpu.VMEM((SC_M,), jnp.int32),
            pltpu.VMEM((NSUB, SC_M), jnp.int32),
            pltpu.VMEM_SHARED((NSUB, SC_M), jnp.int32),
            pltpu.VMEM_SHARED((SC_N,), jnp.int32),
            pltpu.SemaphoreType.DMA,
        ],
    )(_routing_body)


def kernel(qkv, sim, x_size, proj_w, proj_b, logit_scale):
    b, n, c3 = qkv.shape
    ng = n // GS
    routing = make_routing()
    sort_idx, rank = routing(sim.reshape(b, n * SC_M))
    shuffled = jnp.take_along_axis(qkv, sort_idx[:, :, None], axis=1)
    grouped = shuffled.reshape(b * ng, GS, c3)
    out = _grouped_attention(grouped, proj_w, proj_b, logit_scale)
    out = out.reshape(b, n, DIM)
    x = jnp.take_along_axis(out, rank[:, :, None], axis=1)
    return x


# speedup vs baseline: 1.8307x; 1.1012x over previous
import functools

import jax
import jax.numpy as jnp
from jax.experimental import pallas as pl
from jax import lax
from jax.experimental.pallas import tpu as pltpu
from jax.experimental.pallas import tpu_sc as plsc

DIM = 256
NUM_HEADS = 8
HEAD_DIM = DIM // NUM_HEADS
GS = 128
GB = 16
LOG2E = 1.4426950408889634


def _attn_body(scale_ref, qkv_ref, w_ref, b_ref, out_ref):
    scale = jnp.exp(jnp.minimum(scale_ref[0, 0], jnp.log(1.0 / 0.01)))
    sl2 = scale * LOG2E
    wb = w_ref[...].astype(jnp.bfloat16)
    bias = b_ref[...]
    for g in range(GB):
        blk = qkv_ref[g]
        q = blk[:, :DIM]
        k = blk[:, DIM:2 * DIM]
        v = blk[:, 2 * DIM:]
        outs = []
        for h in range(NUM_HEADS):
            sl = slice(h * HEAD_DIM, (h + 1) * HEAD_DIM)
            s = jax.lax.dot_general(q[:, sl], k[:, sl], (((1,), (1,)), ((), ())),
                                    preferred_element_type=jnp.float32) * sl2
            m = jnp.max(s, axis=-1, keepdims=True)
            e = jnp.exp2(s - m)
            r = 1.0 / jnp.sum(e, axis=-1, keepdims=True)
            acc = jax.lax.dot_general(e.astype(jnp.bfloat16),
                                      v[:, sl].astype(jnp.bfloat16),
                                      (((1,), (0,)), ((), ())),
                                      preferred_element_type=jnp.float32)
            outs.append(acc * r)
        o = jnp.concatenate(outs, axis=1)
        res = jax.lax.dot_general(o.astype(jnp.bfloat16), wb,
                                  (((1,), (1,)), ((), ())),
                                  preferred_element_type=jnp.float32)
        out_ref[g] = res + bias


def _grouped_attention(grouped, proj_w, proj_b, logit_scale):
    nblk = grouped.shape[0]
    return pl.pallas_call(
        _attn_body,
        grid=(nblk // GB,),
        in_specs=[
            pl.BlockSpec(memory_space=pltpu.SMEM),
            pl.BlockSpec((GB, GS, 3 * DIM), lambda i: (i, 0, 0)),
            pl.BlockSpec((DIM, DIM), lambda i: (0, 0)),
            pl.BlockSpec((1, DIM), lambda i: (0, 0)),
        ],
        out_specs=pl.BlockSpec((GB, GS, DIM), lambda i: (i, 0, 0)),
        out_shape=jax.ShapeDtypeStruct((nblk, GS, DIM), jnp.float32),
        compiler_params=pltpu.CompilerParams(
            dimension_semantics=("arbitrary",)),
    )(logit_scale, grouped, proj_w, proj_b.reshape(1, DIM))


SC_B = 2
SC_N = 16384
SC_M = 64
NSUB = 16
TPW = SC_N // NSUB
CH = 256
NVR = TPW // 16

RANK_BASE = 1
GCH = 64


def _routing_body(sim_hbm, qkv_hbm, sortidx_hbm, rank_hbm, grouped_hbm,
                  simv, idsv, posv, pos2v, valv, zerov, histv, runv, allhv,
                  gidxv, grow0,
                  histgrid, sortshared, sem, gsem):
    c = lax.axis_index("c")
    s = lax.axis_index("s")
    base = s * TPW

    def chunk_body(ci, _):
        pltpu.sync_copy(sim_hbm.at[c, pl.ds((base + ci * CH) * SC_M, CH * SC_M)], simv)

        def tok_body(t0, _):
            rowbase = (lax.iota(jnp.int32, 16) + t0 * 16) * SC_M
            init = (jnp.full((16,), -jnp.inf, jnp.float32),
                    jnp.zeros((16,), jnp.int32))

            def m_body(m, carry):
                best, bid = carry
                val = plsc.load_gather(simv, [rowbase + m])
                upd = val > best
                return (jnp.where(upd, val, best), jnp.where(upd, m, bid))

            best, bid = lax.fori_loop(0, SC_M, m_body, init)
            idsv[pl.ds(ci * CH + t0 * 16, 16)] = bid
            return _

        return lax.fori_loop(0, CH // 16, tok_body, _)

    lax.fori_loop(0, TPW // CH, chunk_body, None)

    for j in range(4):
        histv[pl.ds(j * 16, 16)] = jnp.zeros((16,), jnp.int32)

    def h_body(i, _):
        ids = idsv[pl.ds(i * 16, 16)]
        r, last = plsc.scan_count(ids)
        r = r - RANK_BASE
        cur = plsc.load_gather(histv, [ids])
        plsc.store_scatter(histv, [ids], cur + r + 1, mask=last)
        return _

    lax.fori_loop(0, NVR, h_body, None)

    pltpu.sync_copy(histv, histgrid.at[s])
    plsc.subcore_barrier()
    pltpu.sync_copy(histgrid, allhv)

    carry = jnp.zeros((), jnp.int32)
    for j in range(4):
        tot = jnp.zeros((16,), jnp.int32)
        mybefore = jnp.zeros((16,), jnp.int32)
        for sp in range(NSUB):
            h = allhv[sp, pl.ds(j * 16, 16)]
            tot = tot + h
            mybefore = mybefore + jnp.where(sp < s, h, 0)
        csum = plsc.cumsum(tot)
        excl = csum - tot + carry
        runv[pl.ds(j * 16, 16)] = excl + mybefore
        carry = carry + jnp.sum(tot)

    def p_body(i, _):
        ids = idsv[pl.ds(i * 16, 16)]
        r, last = plsc.scan_count(ids)
        r = r - RANK_BASE
        cur = plsc.load_gather(runv, [ids])
        pos = cur + r
        posv[pl.ds(i * 16, 16)] = pos
        row = i // (128 // 16)
        col = (i % (128 // 16)) * 16
        pos2v[row, pl.ds(col, 16)] = pos
        valv[pl.ds(i * 16, 16)] = lax.iota(jnp.int32, 16) + (base + i * 16)
        plsc.store_scatter(runv, [ids], cur + r + 1, mask=last)
        return _

    lax.fori_loop(0, NVR, p_body, None)

    pltpu.sync_copy(posv, rank_hbm.at[c, pl.ds(base, TPW)])

    def z_body(i, _):
        zerov[pl.ds(i * 16, 16)] = jnp.zeros((16,), jnp.int32)
        return _

    lax.fori_loop(0, NVR, z_body, None)
    pltpu.sync_copy(zerov, sortshared.at[pl.ds(s * TPW, TPW)])
    plsc.subcore_barrier()
    for j in range(TPW // 128):
        pltpu.async_copy(
            valv.at[pl.ds(j * 128, 128)],
            sortshared.at[pos2v.at[j]],
            sem,
            add=True,
        ).wait()
    plsc.subcore_barrier()
    pltpu.sync_copy(sortshared.at[pl.ds(s * TPW, TPW)],
                    sortidx_hbm.at[c, pl.ds(base, TPW)])

    pltpu.sync_copy(sortshared.at[pl.ds(s * TPW, TPW)], gidxv)

    def gofs_body(i, _):
        gidxv[pl.ds(i * 16, 16)] = gidxv[pl.ds(i * 16, 16)] + c * SC_N
        return _

    lax.fori_loop(0, NVR, gofs_body, None)
    gbase = c * SC_N + s * TPW

    def g_body(j, _):
        pltpu.async_copy(qkv_hbm.at[gidxv.at[pl.ds(j * GCH, GCH)]],
                         grow0, gsem).wait()
        pltpu.sync_copy(grow0, grouped_hbm.at[pl.ds(gbase + j * GCH, GCH)])
        return _

    lax.fori_loop(0, TPW // GCH, g_body, None)


def make_routing():
    mesh = plsc.VectorSubcoreMesh(core_axis_name="c", subcore_axis_name="s")
    return functools.partial(
        pl.kernel,
        out_type=(jax.ShapeDtypeStruct((SC_B, SC_N), jnp.int32),
                  jax.ShapeDtypeStruct((SC_B, SC_N), jnp.int32),
                  jax.ShapeDtypeStruct((SC_B * SC_N, 768), jnp.float32)),
        mesh=mesh,
        compiler_params=pltpu.CompilerParams(needs_layout_passes=False),
        scratch_types=[
            pltpu.VMEM((CH * SC_M,), jnp.float32),
            pltpu.VMEM((TPW,), jnp.int32),
            pltpu.VMEM((TPW,), jnp.int32),
            pltpu.VMEM((TPW // 128, 128), jnp.int32),
            pltpu.VMEM((TPW,), jnp.int32),
            pltpu.VMEM((TPW,), jnp.int32),
            pltpu.VMEM((SC_M,), jnp.int32),
            pltpu.VMEM((SC_M,), jnp.int32),
            pltpu.VMEM((NSUB, SC_M), jnp.int32),
            pltpu.VMEM((TPW,), jnp.int32),
            pltpu.VMEM((GCH, 768), jnp.float32),
            pltpu.VMEM_SHARED((NSUB, SC_M), jnp.int32),
            pltpu.VMEM_SHARED((SC_N,), jnp.int32),
            pltpu.SemaphoreType.DMA,
            pltpu.SemaphoreType.DMA,
        ],
    )(_routing_body)


def kernel(qkv, sim, x_size, proj_w, proj_b, logit_scale):
    b, n, c3 = qkv.shape
    ng = n // GS
    routing = make_routing()
    sort_idx, rank, grouped2 = routing(sim.reshape(b, n * SC_M),
                                       qkv.reshape(b * n, c3))
    grouped = grouped2.reshape(b * ng, GS, c3)
    out = _grouped_attention(grouped, proj_w, proj_b, logit_scale)
    out = out.reshape(b, n, DIM)
    x = jnp.take_along_axis(out, rank[:, :, None], axis=1)
    return x


# speedup vs baseline: 1.8848x; 1.0295x over previous
import functools

import jax
import jax.numpy as jnp
from jax.experimental import pallas as pl
from jax import lax
from jax.experimental.pallas import tpu as pltpu
from jax.experimental.pallas import tpu_sc as plsc

DIM = 256
NUM_HEADS = 8
HEAD_DIM = DIM // NUM_HEADS
GS = 128
GB = 16
LOG2E = 1.4426950408889634


GCH2 = 128


def _unsort_body(rows_hbm, rank_hbm, x_hbm, ridxv, rbuf, sem):
    c = lax.axis_index("c")
    s = lax.axis_index("s")
    base = s * TPW
    pltpu.sync_copy(rank_hbm.at[c, pl.ds(base, TPW)], ridxv)

    def ofs_body(i, _):
        ridxv[pl.ds(i * 16, 16)] = ridxv[pl.ds(i * 16, 16)] + c * SC_N
        return _

    lax.fori_loop(0, NVR, ofs_body, None)

    def g_body(j, _):
        pltpu.async_copy(rows_hbm.at[ridxv.at[pl.ds(j * GCH2, GCH2)]],
                         rbuf, sem).wait()
        pltpu.sync_copy(rbuf,
                        x_hbm.at[pl.ds(c * SC_N + base + j * GCH2, GCH2)])
        return _

    lax.fori_loop(0, TPW // GCH2, g_body, None)


def make_unsort():
    mesh = plsc.VectorSubcoreMesh(core_axis_name="c", subcore_axis_name="s")
    return functools.partial(
        pl.kernel,
        out_type=jax.ShapeDtypeStruct((SC_B * SC_N, DIM), jnp.float32),
        mesh=mesh,
        compiler_params=pltpu.CompilerParams(needs_layout_passes=False),
        scratch_types=[
            pltpu.VMEM((TPW,), jnp.int32),
            pltpu.VMEM((GCH2, DIM), jnp.float32),
            pltpu.SemaphoreType.DMA,
        ],
    )(_unsort_body)


def _attn_body(scale_ref, qkv_ref, w_ref, b_ref, out_ref):
    scale = jnp.exp(jnp.minimum(scale_ref[0, 0], jnp.log(1.0 / 0.01)))
    sl2 = scale * LOG2E
    wb = w_ref[...].astype(jnp.bfloat16)
    bias = b_ref[...]
    for g in range(GB):
        blk = qkv_ref[g]
        q = blk[:, :DIM]
        k = blk[:, DIM:2 * DIM]
        v = blk[:, 2 * DIM:]
        outs = []
        for h in range(NUM_HEADS):
            sl = slice(h * HEAD_DIM, (h + 1) * HEAD_DIM)
            s = jax.lax.dot_general(q[:, sl], k[:, sl], (((1,), (1,)), ((), ())),
                                    preferred_element_type=jnp.float32) * sl2
            m = jnp.max(s, axis=-1, keepdims=True)
            e = jnp.exp2(s - m)
            r = 1.0 / jnp.sum(e, axis=-1, keepdims=True)
            acc = jax.lax.dot_general(e.astype(jnp.bfloat16),
                                      v[:, sl].astype(jnp.bfloat16),
                                      (((1,), (0,)), ((), ())),
                                      preferred_element_type=jnp.float32)
            outs.append(acc * r)
        o = jnp.concatenate(outs, axis=1)
        res = jax.lax.dot_general(o.astype(jnp.bfloat16), wb,
                                  (((1,), (1,)), ((), ())),
                                  preferred_element_type=jnp.float32)
        out_ref[g] = res + bias


def _grouped_attention(grouped, proj_w, proj_b, logit_scale):
    nblk = grouped.shape[0]
    return pl.pallas_call(
        _attn_body,
        grid=(nblk // GB,),
        in_specs=[
            pl.BlockSpec(memory_space=pltpu.SMEM),
            pl.BlockSpec((GB, GS, 3 * DIM), lambda i: (i, 0, 0)),
            pl.BlockSpec((DIM, DIM), lambda i: (0, 0)),
            pl.BlockSpec((1, DIM), lambda i: (0, 0)),
        ],
        out_specs=pl.BlockSpec((GB, GS, DIM), lambda i: (i, 0, 0)),
        out_shape=jax.ShapeDtypeStruct((nblk, GS, DIM), jnp.float32),
        compiler_params=pltpu.CompilerParams(
            dimension_semantics=("arbitrary",)),
    )(logit_scale, grouped, proj_w, proj_b.reshape(1, DIM))


SC_B = 2
SC_N = 16384
SC_M = 64
NSUB = 16
TPW = SC_N // NSUB
CH = 256
NVR = TPW // 16

RANK_BASE = 1
GCH = 64


def _routing_body(sim_hbm, qkv_hbm, sortidx_hbm, rank_hbm, grouped_hbm,
                  simv, idsv, posv, pos2v, valv, zerov, histv, runv, allhv,
                  gidxv, grow0,
                  histgrid, sortshared, sem, gsem):
    c = lax.axis_index("c")
    s = lax.axis_index("s")
    base = s * TPW

    def chunk_body(ci, _):
        pltpu.sync_copy(sim_hbm.at[c, pl.ds((base + ci * CH) * SC_M, CH * SC_M)], simv)

        def tok_body(t0, _):
            rowbase = (lax.iota(jnp.int32, 16) + t0 * 16) * SC_M
            init = (jnp.full((16,), -jnp.inf, jnp.float32),
                    jnp.zeros((16,), jnp.int32))

            def m_body(m, carry):
                best, bid = carry
                val = plsc.load_gather(simv, [rowbase + m])
                upd = val > best
                return (jnp.where(upd, val, best), jnp.where(upd, m, bid))

            best, bid = lax.fori_loop(0, SC_M, m_body, init)
            idsv[pl.ds(ci * CH + t0 * 16, 16)] = bid
            return _

        return lax.fori_loop(0, CH // 16, tok_body, _)

    lax.fori_loop(0, TPW // CH, chunk_body, None)

    for j in range(4):
        histv[pl.ds(j * 16, 16)] = jnp.zeros((16,), jnp.int32)

    def h_body(i, _):
        ids = idsv[pl.ds(i * 16, 16)]
        r, last = plsc.scan_count(ids)
        r = r - RANK_BASE
        cur = plsc.load_gather(histv, [ids])
        plsc.store_scatter(histv, [ids], cur + r + 1, mask=last)
        return _

    lax.fori_loop(0, NVR, h_body, None)

    pltpu.sync_copy(histv, histgrid.at[s])
    plsc.subcore_barrier()
    pltpu.sync_copy(histgrid, allhv)

    carry = jnp.zeros((), jnp.int32)
    for j in range(4):
        tot = jnp.zeros((16,), jnp.int32)
        mybefore = jnp.zeros((16,), jnp.int32)
        for sp in range(NSUB):
            h = allhv[sp, pl.ds(j * 16, 16)]
            tot = tot + h
            mybefore = mybefore + jnp.where(sp < s, h, 0)
        csum = plsc.cumsum(tot)
        excl = csum - tot + carry
        runv[pl.ds(j * 16, 16)] = excl + mybefore
        carry = carry + jnp.sum(tot)

    def p_body(i, _):
        ids = idsv[pl.ds(i * 16, 16)]
        r, last = plsc.scan_count(ids)
        r = r - RANK_BASE
        cur = plsc.load_gather(runv, [ids])
        pos = cur + r
        posv[pl.ds(i * 16, 16)] = pos
        row = i // (128 // 16)
        col = (i % (128 // 16)) * 16
        pos2v[row, pl.ds(col, 16)] = pos
        valv[pl.ds(i * 16, 16)] = lax.iota(jnp.int32, 16) + (base + i * 16)
        plsc.store_scatter(runv, [ids], cur + r + 1, mask=last)
        return _

    lax.fori_loop(0, NVR, p_body, None)

    pltpu.sync_copy(posv, rank_hbm.at[c, pl.ds(base, TPW)])

    def z_body(i, _):
        zerov[pl.ds(i * 16, 16)] = jnp.zeros((16,), jnp.int32)
        return _

    lax.fori_loop(0, NVR, z_body, None)
    pltpu.sync_copy(zerov, sortshared.at[pl.ds(s * TPW, TPW)])
    plsc.subcore_barrier()
    for j in range(TPW // 128):
        pltpu.async_copy(
            valv.at[pl.ds(j * 128, 128)],
            sortshared.at[pos2v.at[j]],
            sem,
            add=True,
        ).wait()
    plsc.subcore_barrier()
    pltpu.sync_copy(sortshared.at[pl.ds(s * TPW, TPW)],
                    sortidx_hbm.at[c, pl.ds(base, TPW)])

    pltpu.sync_copy(sortshared.at[pl.ds(s * TPW, TPW)], gidxv)

    def gofs_body(i, _):
        gidxv[pl.ds(i * 16, 16)] = gidxv[pl.ds(i * 16, 16)] + c * SC_N
        return _

    lax.fori_loop(0, NVR, gofs_body, None)
    gbase = c * SC_N + s * TPW

    def g_body(j, _):
        pltpu.async_copy(qkv_hbm.at[gidxv.at[pl.ds(j * GCH, GCH)]],
                         grow0, gsem).wait()
        pltpu.sync_copy(grow0, grouped_hbm.at[pl.ds(gbase + j * GCH, GCH)])
        return _

    lax.fori_loop(0, TPW // GCH, g_body, None)


def make_routing():
    mesh = plsc.VectorSubcoreMesh(core_axis_name="c", subcore_axis_name="s")
    return functools.partial(
        pl.kernel,
        out_type=(jax.ShapeDtypeStruct((SC_B, SC_N), jnp.int32),
                  jax.ShapeDtypeStruct((SC_B, SC_N), jnp.int32),
                  jax.ShapeDtypeStruct((SC_B * SC_N, 768), jnp.float32)),
        mesh=mesh,
        compiler_params=pltpu.CompilerParams(needs_layout_passes=False),
        scratch_types=[
            pltpu.VMEM((CH * SC_M,), jnp.float32),
            pltpu.VMEM((TPW,), jnp.int32),
            pltpu.VMEM((TPW,), jnp.int32),
            pltpu.VMEM((TPW // 128, 128), jnp.int32),
            pltpu.VMEM((TPW,), jnp.int32),
            pltpu.VMEM((TPW,), jnp.int32),
            pltpu.VMEM((SC_M,), jnp.int32),
            pltpu.VMEM((SC_M,), jnp.int32),
            pltpu.VMEM((NSUB, SC_M), jnp.int32),
            pltpu.VMEM((TPW,), jnp.int32),
            pltpu.VMEM((GCH, 768), jnp.float32),
            pltpu.VMEM_SHARED((NSUB, SC_M), jnp.int32),
            pltpu.VMEM_SHARED((SC_N,), jnp.int32),
            pltpu.SemaphoreType.DMA,
            pltpu.SemaphoreType.DMA,
        ],
    )(_routing_body)


def kernel(qkv, sim, x_size, proj_w, proj_b, logit_scale):
    b, n, c3 = qkv.shape
    ng = n // GS
    routing = make_routing()
    sort_idx, rank, grouped2 = routing(sim.reshape(b, n * SC_M),
                                       qkv.reshape(b * n, c3))
    grouped = grouped2.reshape(b * ng, GS, c3)
    out = _grouped_attention(grouped, proj_w, proj_b, logit_scale)
    unsort = make_unsort()
    x = unsort(out.reshape(b * n, DIM), rank)
    return x.reshape(b, n, DIM)
